# baseline (speedup 1.0000x reference)
import functools

import jax
import jax.numpy as jnp
from jax import lax
from jax.experimental import pallas as pl
from jax.experimental.pallas import tpu as pltpu
from jax.experimental.pallas import tpu_sc as plsc

_BC = 1024


def _relayout_body(x_ref, o_ref):
    xt = x_ref[...].T
    o_ref[...] = jnp.concatenate([xt[: _BC // 2], xt[_BC // 2 :]], axis=1)


@functools.cache
def _make_relayout(V, D):
    nblk = pl.cdiv(V, _BC)
    return pl.pallas_call(
        _relayout_body,
        grid=(nblk,),
        in_specs=[pl.BlockSpec((D, _BC), lambda c: (0, c))],
        out_specs=pl.BlockSpec((_BC // 2, 2 * D), lambda c: (c, 0)),
        out_shape=jax.ShapeDtypeStruct((nblk * _BC // 2, 2 * D), jnp.float32),
    )


@functools.cache
def _make_gather(H, BT, V2, D):
    info = plsc.get_sparse_core_info()
    NC, NS = info.num_cores, info.num_subcores
    NW = NC * NS
    CB = 256
    n_blocks = H * BT * 128 // CB
    assert n_blocks % NW == 0
    blk_per_w = n_blocks // NW
    bph = (BT * 128) // CB
    idx_per_w = blk_per_w * CB

    mesh = plsc.VectorSubcoreMesh(core_axis_name="c", subcore_axis_name="s")

    @functools.partial(
        pl.kernel,
        mesh=mesh,
        compiler_params=pltpu.CompilerParams(
            use_tc_tiling_on_sc=False, needs_layout_passes=False
        ),
        out_type=jax.ShapeDtypeStruct((H, D // 8, BT, 8, 128), jnp.float32),
        scratch_types=[
            pltpu.VMEM((idx_per_w,), jnp.int32),
            pltpu.VMEM((2, CB, D), jnp.float32),
            pltpu.VMEM((2, D // 8, CB // 128, 8, 128), jnp.float32),
            [pltpu.SemaphoreType.DMA] * 2,
            [pltpu.SemaphoreType.DMA] * 2,
        ],
    )
    def gather_kernel(idx_hbm, table_hbm, out_hbm, idx_v, rows_v, rowst_v, gsems, osems):
        wid = lax.axis_index("s") * NC + lax.axis_index("c")
        blk0 = wid * blk_per_w

        pltpu.sync_copy(idx_hbm.at[pl.ds(blk0 * CB, idx_per_w)], idx_v)

        def gather_dma(k, s):
            return pltpu.make_async_copy(
                table_hbm.at[idx_v.at[pl.ds(k * CB, CB)]],
                rows_v.at[s],
                gsems[s],
            )

        def store_dma(k, s):
            blk = blk0 + k
            h = blk // bph
            c = blk - h * bph
            return pltpu.make_async_copy(
                rowst_v.at[s],
                out_hbm.at[h, :, pl.ds(c * (CB // 128), CB // 128), :, :],
                osems[s],
            )

        iota16 = lax.iota(jnp.int32, 16)

        def transpose_block(s):
            def per_d(d, carry):
                dt = d // 8
                ds_ = d - dt * 8
                dvec = jnp.full((16,), d, jnp.int32)
                for j in range(CB // 16):
                    src = plsc.load_gather(
                        rows_v.at[s], [iota16 + (j * 16), dvec]
                    )
                    rowst_v[s, dt, (j * 16) // 128, ds_, pl.ds((j * 16) % 128, 16)] = src
                return carry

            lax.fori_loop(0, D, per_d, 0)

        gather_dma(0, 0).start()

        def body(g, carry):
            for s in (0, 1):
                k = 2 * g + s
                gather_dma(k, s).wait()

                @pl.when(k + 1 < blk_per_w)
                def _():
                    gather_dma(k + 1, 1 - s).start()

                @pl.when(k >= 2)
                def _():
                    store_dma(k - 2, s).wait()

                transpose_block(s)
                store_dma(k, s).start()
            return carry

        assert blk_per_w % 2 == 0
        lax.fori_loop(0, blk_per_w // 2, body, 0)

        store_dma(blk_per_w - 2, 0).wait()
        store_dma(blk_per_w - 1, 1).wait()

    return gather_kernel


def kernel(goal_encoding, embed_table):
    batch, hist = goal_encoding.shape
    v, d = embed_table.shape
    i = goal_encoding.T.astype(jnp.int32)
    idx = (i & ~(_BC - 1)) + ((i & (_BC // 2 - 1)) << 1) + ((i >> 9) & 1)
    idx = idx.reshape(-1)
    lin = _make_relayout(v, d)(embed_table.T)
    v2 = 2 * lin.shape[0]
    table_lin = lin.reshape(v2, d)
    bt = batch // 128
    z = _make_gather(hist, bt, v2, d)(idx, table_lin)
    out = z.transpose(2, 4, 0, 1, 3).reshape(batch, hist, d)
    return out


# speedup vs baseline: 1.2249x; 1.2249x over previous
import functools

import jax
import jax.numpy as jnp
from jax import lax
from jax.experimental import pallas as pl
from jax.experimental.pallas import tpu as pltpu
from jax.experimental.pallas import tpu_sc as plsc

_BC = 1024


def _relayout_body(x_ref, o_ref):
    x = x_ref[...]
    eye = jnp.eye(x.shape[0], dtype=x.dtype)
    xt = lax.dot_general(
        x, eye, (((0,), (0,)), ((), ())), preferred_element_type=jnp.float32
    )
    o_ref[...] = jnp.concatenate([xt[: _BC // 2], xt[_BC // 2 :]], axis=1)


@functools.cache
def _make_relayout(V, D):
    nblk = pl.cdiv(V, _BC)
    return pl.pallas_call(
        _relayout_body,
        grid=(nblk,),
        in_specs=[pl.BlockSpec((D, _BC), lambda c: (0, c))],
        out_specs=pl.BlockSpec((_BC // 2, 2 * D), lambda c: (c, 0)),
        out_shape=jax.ShapeDtypeStruct((nblk * _BC // 2, 2 * D), jnp.float32),
    )


def _outx_body(x_ref, o_ref):
    x = x_ref[...][0]
    eye = jnp.eye(x.shape[1], dtype=x.dtype)
    o_ref[...] = lax.dot_general(
        eye, x, (((1,), (1,)), ((), ())), preferred_element_type=jnp.float32
    )[None]


@functools.cache
def _make_outx(H, B, D, BB=2048):
    return pl.pallas_call(
        _outx_body,
        grid=(H, B // BB),
        in_specs=[pl.BlockSpec((1, BB, D), lambda h, b: (h, b, 0))],
        out_specs=pl.BlockSpec((1, D, BB), lambda h, b: (h, 0, b)),
        out_shape=jax.ShapeDtypeStruct((H, D, B), jnp.float32),
    )


@functools.cache
def _make_gather(B, V, D):
    info = plsc.get_sparse_core_info()
    NC, NS = info.num_cores, info.num_subcores
    NW = NC * NS
    assert B % NW == 0
    b_per_w = B // NW
    C = 512
    NBUF = 2
    assert b_per_w % (C * NBUF) == 0
    n_chunks = b_per_w // C
    n_groups = n_chunks // NBUF

    mesh = plsc.VectorSubcoreMesh(core_axis_name="c", subcore_axis_name="s")

    @functools.partial(
        pl.kernel,
        mesh=mesh,
        compiler_params=pltpu.CompilerParams(use_tc_tiling_on_sc=False),
        out_type=jax.ShapeDtypeStruct((B, D), jnp.float32),
        scratch_types=[
            pltpu.VMEM((b_per_w,), jnp.int32),
            pltpu.VMEM((NBUF, C, D), jnp.float32),
            [pltpu.SemaphoreType.DMA] * NBUF,
            [pltpu.SemaphoreType.DMA] * NBUF,
        ],
    )
    def gather_kernel(idx_hbm, table_hbm, out_hbm, idx_v, rows_v, gsems, osems):
        wid = lax.axis_index("s") * NC + lax.axis_index("c")
        base = wid * b_per_w

        pltpu.sync_copy(idx_hbm.at[pl.ds(base, b_per_w)], idx_v)

        def gather_dma(i, b):
            return pltpu.make_async_copy(
                table_hbm.at[idx_v.at[pl.ds(i * C, C)]], rows_v.at[b], gsems[b]
            )

        def store_dma(i, b):
            return pltpu.make_async_copy(
                rows_v.at[b], out_hbm.at[pl.ds(base + i * C, C)], osems[b]
            )

        for b in range(NBUF):
            gather_dma(b, b).start()

        def group(g, carry):
            for b in range(NBUF):
                i = g * NBUF + b
                gather_dma(i, b).wait()
                store_dma(i, b).start()
            for b in range(NBUF):
                i = g * NBUF + b
                store_dma(i, b).wait()
                gather_dma(i + NBUF, b).start()
            return carry

        lax.fori_loop(0, n_groups - 1, group, 0)

        for b in range(NBUF):
            i = (n_groups - 1) * NBUF + b
            gather_dma(i, b).wait()
            store_dma(i, b).start()
        for b in range(NBUF):
            i = (n_groups - 1) * NBUF + b
            store_dma(i, b).wait()

    return gather_kernel


def kernel(goal_encoding, embed_table):
    batch, hist = goal_encoding.shape
    v, d = embed_table.shape
    i = goal_encoding.T.astype(jnp.int32)
    idx = (i & ~(_BC - 1)) + ((i & (_BC // 2 - 1)) << 1) + ((i >> 9) & 1)
    idx = idx.reshape(-1)
    lin = _make_relayout(v, d)(embed_table.T)
    v2 = 2 * lin.shape[0]
    table_lin = lin.reshape(v2, d)
    g = _make_gather(batch * hist, v2, d)(idx, table_lin)
    g3 = g.reshape(hist, batch, d)
    z = _make_outx(hist, batch, d)(g3)
    return z.transpose(2, 0, 1)


# speedup vs baseline: 1.5578x; 1.2718x over previous
import functools

import jax
import jax.numpy as jnp
from jax import lax
from jax.experimental import pallas as pl
from jax.experimental.pallas import tpu as pltpu
from jax.experimental.pallas import tpu_sc as plsc

_BC = 1024


def _relayout_body(x_ref, o_ref):
    xt = x_ref[...].T
    o_ref[...] = jnp.concatenate([xt[: _BC // 2], xt[_BC // 2 :]], axis=1)


@functools.cache
def _make_relayout(V, D):
    nblk = pl.cdiv(V, _BC)
    return pl.pallas_call(
        _relayout_body,
        grid=(nblk,),
        in_specs=[pl.BlockSpec((D, _BC), lambda c: (0, c))],
        out_specs=pl.BlockSpec((_BC // 2, 2 * D), lambda c: (c, 0)),
        out_shape=jax.ShapeDtypeStruct((nblk * _BC // 2, 2 * D), jnp.float32),
    )


def _outx_body(x_ref, o_ref):
    x = x_ref[...][0]
    d = o_ref.shape[1]
    o_ref[...] = x[:, :d].T[None]


@functools.cache
def _make_outx(H, B, D, BB=2048):
    return pl.pallas_call(
        _outx_body,
        grid=(H, B // BB),
        in_specs=[pl.BlockSpec((1, BB, 2 * D), lambda h, b: (h, b, 0))],
        out_specs=pl.BlockSpec((1, D, BB), lambda h, b: (h, 0, b)),
        out_shape=jax.ShapeDtypeStruct((H, D, B), jnp.float32),
    )


@functools.cache
def _make_gather(B, V, D):
    info = plsc.get_sparse_core_info()
    NC, NS = info.num_cores, info.num_subcores
    NW = NC * NS
    assert B % NW == 0
    b_per_w = B // NW
    C = 512
    NBUF = 2
    assert b_per_w % (C * NBUF) == 0
    n_chunks = b_per_w // C
    n_groups = n_chunks // NBUF

    mesh = plsc.VectorSubcoreMesh(core_axis_name="c", subcore_axis_name="s")

    @functools.partial(
        pl.kernel,
        mesh=mesh,
        compiler_params=pltpu.CompilerParams(use_tc_tiling_on_sc=False),
        out_type=jax.ShapeDtypeStruct((B, 2 * D), jnp.float32),
        scratch_types=[
            pltpu.VMEM((b_per_w,), jnp.int32),
            pltpu.VMEM((NBUF, C, D), jnp.float32),
            [pltpu.SemaphoreType.DMA] * NBUF,
            [pltpu.SemaphoreType.DMA] * NBUF,
        ],
    )
    def gather_kernel(idx_hbm, table_hbm, out_hbm, idx_v, rows_v, gsems, osems):
        wid = lax.axis_index("s") * NC + lax.axis_index("c")
        base = wid * b_per_w

        pltpu.sync_copy(idx_hbm.at[pl.ds(base, b_per_w)], idx_v)

        def gather_dma(i, b):
            return pltpu.make_async_copy(
                table_hbm.at[idx_v.at[pl.ds(i * C, C)]], rows_v.at[b], gsems[b]
            )

        def store_dma(i, b):
            return pltpu.make_async_copy(
                rows_v.at[b],
                out_hbm.at[pl.ds(base + i * C, C), pl.ds(0, D)],
                osems[b],
            )

        for b in range(NBUF):
            gather_dma(b, b).start()

        def group(g, carry):
            for b in range(NBUF):
                i = g * NBUF + b
                gather_dma(i, b).wait()
                store_dma(i, b).start()
            for b in range(NBUF):
                i = g * NBUF + b
                store_dma(i, b).wait()
                gather_dma(i + NBUF, b).start()
            return carry

        lax.fori_loop(0, n_groups - 1, group, 0)

        for b in range(NBUF):
            i = (n_groups - 1) * NBUF + b
            gather_dma(i, b).wait()
            store_dma(i, b).start()
        for b in range(NBUF):
            i = (n_groups - 1) * NBUF + b
            store_dma(i, b).wait()

    return gather_kernel


def kernel(goal_encoding, embed_table):
    batch, hist = goal_encoding.shape
    v, d = embed_table.shape
    i = goal_encoding.T.astype(jnp.int32)
    idx = (i & ~(_BC - 1)) + ((i & (_BC // 2 - 1)) << 1) + ((i >> 9) & 1)
    idx = idx.reshape(-1)
    lin = _make_relayout(v, d)(embed_table.T)
    v2 = 2 * lin.shape[0]
    table_lin = lin.reshape(v2, d)
    g = _make_gather(batch * hist, v2, d)(idx, table_lin)
    g3 = g.reshape(hist, batch, 2 * d)
    z = _make_outx(hist, batch, d)(g3)
    return z.transpose(2, 0, 1)


# speedup vs baseline: 2.7117x; 1.7407x over previous
import functools

import jax
import jax.numpy as jnp
from jax import lax
from jax.experimental import pallas as pl
from jax.experimental.pallas import tpu as pltpu
from jax.experimental.pallas import tpu_sc as plsc

_BC = 4096
_HS = (_BC // 2).bit_length() - 1


def _relayout_body(x_ref, o_ref):
    xt = x_ref[...].T
    o_ref[...] = jnp.concatenate([xt[: _BC // 2], xt[_BC // 2 :]], axis=1)


@functools.cache
def _make_relayout(V, D):
    nblk = pl.cdiv(V, _BC)
    return pl.pallas_call(
        _relayout_body,
        grid=(nblk,),
        in_specs=[pl.BlockSpec((D, _BC), lambda c: (0, c))],
        out_specs=pl.BlockSpec((_BC // 2, 2 * D), lambda c: (c, 0)),
        out_shape=jax.ShapeDtypeStruct((nblk * _BC // 2, 2 * D), jnp.float32),
    )


def _outx_body(x_ref, o_ref):
    x = x_ref[...][0]
    d = o_ref.shape[1]
    o_ref[...] = x[:, :d].T[None]


@functools.cache
def _make_outx(H, B, D, BB=8192):
    return pl.pallas_call(
        _outx_body,
        grid=(H, B // BB),
        in_specs=[pl.BlockSpec((1, BB, 2 * D), lambda h, b: (h, b, 0))],
        out_specs=pl.BlockSpec((1, D, BB), lambda h, b: (h, 0, b)),
        out_shape=jax.ShapeDtypeStruct((H, D, B), jnp.float32),
    )


@functools.cache
def _make_gather(B, V, D):
    info = plsc.get_sparse_core_info()
    NC, NS = info.num_cores, info.num_subcores
    NW = NC * NS
    assert B % NW == 0
    b_per_w = B // NW
    C = 512
    NBUF = 2
    assert b_per_w % (C * NBUF) == 0
    n_chunks = b_per_w // C
    n_groups = n_chunks // NBUF

    mesh = plsc.VectorSubcoreMesh(core_axis_name="c", subcore_axis_name="s")

    @functools.partial(
        pl.kernel,
        mesh=mesh,
        compiler_params=pltpu.CompilerParams(use_tc_tiling_on_sc=False),
        out_type=jax.ShapeDtypeStruct((B, 2 * D), jnp.float32),
        scratch_types=[
            pltpu.VMEM((b_per_w,), jnp.int32),
            pltpu.VMEM((NBUF, C, D), jnp.float32),
            [pltpu.SemaphoreType.DMA] * NBUF,
            [pltpu.SemaphoreType.DMA] * NBUF,
        ],
    )
    def gather_kernel(idx_hbm, table_hbm, out_hbm, idx_v, rows_v, gsems, osems):
        wid = lax.axis_index("s") * NC + lax.axis_index("c")
        base = wid * b_per_w

        pltpu.sync_copy(idx_hbm.at[pl.ds(base, b_per_w)], idx_v)

        def gather_dma(i, b):
            return pltpu.make_async_copy(
                table_hbm.at[idx_v.at[pl.ds(i * C, C)]], rows_v.at[b], gsems[b]
            )

        def store_dma(i, b):
            return pltpu.make_async_copy(
                rows_v.at[b],
                out_hbm.at[pl.ds(base + i * C, C), pl.ds(0, D)],
                osems[b],
            )

        for b in range(NBUF):
            gather_dma(b, b).start()

        def group(g, carry):
            for b in range(NBUF):
                i = g * NBUF + b
                gather_dma(i, b).wait()
                store_dma(i, b).start()
            for b in range(NBUF):
                i = g * NBUF + b
                store_dma(i, b).wait()
                gather_dma(i + NBUF, b).start()
            return carry

        lax.fori_loop(0, n_groups - 1, group, 0)

        for b in range(NBUF):
            i = (n_groups - 1) * NBUF + b
            gather_dma(i, b).wait()
            store_dma(i, b).start()
        for b in range(NBUF):
            i = (n_groups - 1) * NBUF + b
            store_dma(i, b).wait()

    return gather_kernel


def kernel(goal_encoding, embed_table):
    batch, hist = goal_encoding.shape
    v, d = embed_table.shape
    i = goal_encoding.T.astype(jnp.int32)
    idx = (i & ~(_BC - 1)) + ((i & (_BC // 2 - 1)) << 1) + ((i >> _HS) & 1)
    idx = idx.reshape(-1)
    lin = _make_relayout(v, d)(embed_table.T)
    v2 = 2 * lin.shape[0]
    table_lin = lin.reshape(v2, d)
    g = _make_gather(batch * hist, v2, d)(idx, table_lin)
    g3 = g.reshape(hist, batch, 2 * d)
    z = _make_outx(hist, batch, d)(g3)
    return z.transpose(2, 0, 1)
